# Initial kernel scaffold; baseline (speedup 1.0000x reference)
import jax
import jax.numpy as jnp
from jax.experimental import pallas as pl


def kernel(coordinates, node_attrs, edge_attrs, edge_index, num_nodes, W_node, W_edge, W_cat, b_cat, W_I, b_I, W_A, b_A, W_S, b_S, ln_g, ln_b, W_m1, b_m1, W_m2, b_m2, W_Ia, W_Aa, W_Sa):
    raise NotImplementedError("write your pallas kernel here")


# speedup vs baseline: 15.1725x; 15.1725x over previous
import functools

import jax
import jax.numpy as jnp
import numpy as np
from jax import lax
from jax.experimental import pallas as pl
from jax.experimental.pallas import tpu as pltpu
from jax.experimental.pallas import tpu_sc as plsc

N = 10000
E = 160000
NODE_F = 128
EDGE_F = 16
F = 64
R = 32
CUT = 5.0
BETA = (2.0 / R * (1.0 - float(np.exp(-CUT)))) ** (-2)

NC = 2
NS = 16
NW = NC * NS
CHUNK = 128
NCHUNKS = E // CHUNK
BASE_CH = NCHUNKS // NW
EXTRA = NCHUNKS - BASE_CH * NW
NACC = 10240
NPT = NACC // NS
ZROWS = 128

def _mesh():
    return plsc.VectorSubcoreMesh(core_axis_name="c", subcore_axis_name="s")


GW = 128
PG = 5
PW = 128


def _nf_body(na_ref, w_ref, c_ref, o_ref):
    o_ref[:, 0:F] = jnp.dot(na_ref[...], w_ref[...],
                            preferred_element_type=jnp.float32)
    o_ref[:, F:F + 3] = c_ref[...]
    o_ref[:, F + 3:GW] = jnp.zeros((N, GW - F - 3), jnp.float32)


def _node_feats(node_attrs, w_node, coordinates):
    return pl.pallas_call(
        _nf_body,
        out_shape=jax.ShapeDtypeStruct((N, GW), jnp.float32),
    )(node_attrs, w_node, coordinates)


def _gather_body(nf_hbm, eidx_hbm, gj_hbm, gi_hbm,
                 sbuf, rbuf, jrows, irows, sem1, sem2):
    c = lax.axis_index("c")
    s = lax.axis_index("s")
    w = c * NS + s

    def do_chunk(base):
        pltpu.sync_copy(eidx_hbm.at[0, pl.ds(base, CHUNK)], sbuf)
        pltpu.sync_copy(eidx_hbm.at[1, pl.ds(base, CHUNK)], rbuf)
        cp1 = pltpu.async_copy(nf_hbm.at[rbuf], jrows, sem1)
        cp2 = pltpu.async_copy(nf_hbm.at[sbuf], irows, sem2)
        cp1.wait()
        cp2.wait()
        pltpu.sync_copy(jrows, gj_hbm.at[pl.ds(base, CHUNK)])
        pltpu.sync_copy(irows, gi_hbm.at[pl.ds(base, CHUNK)])

    def chunk_body(i, carry):
        do_chunk((w + NW * i) * CHUNK)
        return carry

    lax.fori_loop(0, BASE_CH, chunk_body, 0)

    @pl.when(w < EXTRA)
    def _():
        do_chunk((w + NW * BASE_CH) * CHUNK)


@functools.lru_cache(maxsize=None)
def _make_gather_k():
    return pl.kernel(
        _gather_body,
        mesh=_mesh(),
        out_type=(
            jax.ShapeDtypeStruct((E, GW), jnp.float32),
            jax.ShapeDtypeStruct((E, GW), jnp.float32),
        ),
        scratch_types=[
            pltpu.VMEM((CHUNK,), jnp.int32),
            pltpu.VMEM((CHUNK,), jnp.int32),
            pltpu.VMEM((CHUNK, GW), jnp.float32),
            pltpu.VMEM((CHUNK, GW), jnp.float32),
            pltpu.SemaphoreType.DMA,
            pltpu.SemaphoreType.DMA,
        ],
    )


TE = 256


def _edge_body(gj_ref, gi_ref, ea_ref, wcj_ref, wci_ref, we_ref,
               wce_ref, bcat_ref, wI_ref, bI_ref, wA_ref, bA_ref, wS_ref,
               bS_ref, mu_ref, p_ref):
    f32 = jnp.float32
    ef = jnp.dot(ea_ref[...], we_ref[...], preferred_element_type=f32)
    nfji = (jnp.dot(gj_ref[:, 0:F], wcj_ref[...], preferred_element_type=f32)
            + jnp.dot(gi_ref[:, 0:F], wci_ref[...], preferred_element_type=f32)
            + jnp.dot(ef, wce_ref[...], preferred_element_type=f32)
            + bcat_ref[...])
    vx = gi_ref[:, F:F + 1] - gj_ref[:, F:F + 1]
    vy = gi_ref[:, F + 1:F + 2] - gj_ref[:, F + 1:F + 2]
    vz = gi_ref[:, F + 2:F + 3] - gj_ref[:, F + 2:F + 3]
    lsq = vx * vx + vy * vy + vz * vz
    l = jnp.sqrt(lsq)
    inv = 1.0 / l
    nx = vx * inv
    ny = vy * inv
    nz = vz * inv
    d = jnp.exp(-l) - mu_ref[...]
    rbf = jnp.exp((-BETA) * d * d)
    phi = jnp.where(l < CUT, 0.5 * (jnp.cos((np.pi / CUT) * l) + 1.0), 0.0)
    q = rbf * phi
    f0I = (jnp.dot(q, wI_ref[...], preferred_element_type=f32) + bI_ref[...]) * phi
    f0A = (jnp.dot(q, wA_ref[...], preferred_element_type=f32) + bA_ref[...]) * phi
    f0S = (jnp.dot(q, wS_ref[...], preferred_element_type=f32) + bS_ref[...]) * phi
    gI = nfji * f0I
    gA = nfji * f0A
    gS = nfji * f0S
    t = 1.0 / 3.0
    m = [
        gI + gS * (nx * nx - t),
        gS * (nx * ny) - gA * nz,
        gS * (nx * nz) + gA * ny,
        gS * (nx * ny) + gA * nz,
        gI + gS * (ny * ny - t),
        gS * (ny * nz) - gA * nx,
        gS * (nx * nz) - gA * ny,
        gS * (ny * nz) + gA * nx,
        gI + gS * (nz * nz - t),
    ]
    for g in range(4):
        p_ref[g, :, 0:F] = m[2 * g]
        p_ref[g, :, F:PW] = m[2 * g + 1]
    p_ref[4, :, 0:F] = m[8]
    p_ref[4, :, F:PW] = jnp.zeros_like(m[8])


def _edge_stage(gj, gi, ea, wcj, wci, w_edge, wce, bcat, wI, bI, wA,
                bA, wS, bS, mu_row):
    n_blocks = E // TE
    full = lambda shape: pl.BlockSpec(shape, lambda i: tuple(0 for _ in shape))
    return pl.pallas_call(
        _edge_body,
        grid=(n_blocks,),
        in_specs=[
            pl.BlockSpec((TE, GW), lambda i: (i, 0)),
            pl.BlockSpec((TE, GW), lambda i: (i, 0)),
            pl.BlockSpec((TE, EDGE_F), lambda i: (i, 0)),
            full((F, F)), full((F, F)), full((EDGE_F, F)), full((F, F)),
            full((1, F)), full((R, F)), full((1, F)), full((R, F)),
            full((1, F)), full((R, F)), full((1, F)), full((1, R)),
        ],
        out_specs=pl.BlockSpec((PG, TE, PW), lambda i: (0, i, 0)),
        out_shape=jax.ShapeDtypeStruct((PG, E, PW), jnp.float32),
    )(gj, gi, ea, wcj, wci, w_edge, wce, bcat, wI, bI, wA, bA, wS, bS,
      mu_row)


def _scatter_body(eidx_hbm, p_hbm, out0_hbm, out1_hbm, ibuf, ubuf, zbuf, acc):
    c = lax.axis_index("c")
    s = lax.axis_index("s")
    w = c * NS + s
    zv = jnp.zeros((16,), jnp.float32)

    def zrow(r, carry):
        def zcol(j, carry2):
            zbuf[r, pl.ds(j * 16, 16)] = zv
            return carry2
        return lax.fori_loop(0, PW // 16, zcol, carry)

    lax.fori_loop(0, ZROWS, zrow, 0)

    for g in range(PG):
        plsc.subcore_barrier()
        for j in range(NPT // ZROWS):
            pltpu.sync_copy(zbuf, acc.at[pl.ds(s * NPT + j * ZROWS, ZROWS)])
        plsc.subcore_barrier()

        def acc_chunk_at(base):
            pltpu.sync_copy(eidx_hbm.at[1, pl.ds(base, CHUNK)], ibuf)
            pltpu.sync_copy(p_hbm.at[g, pl.ds(base, CHUNK)], ubuf)
            pltpu.sync_copy(ubuf, acc.at[ibuf], add=True)

        def acc_chunk(i, carry):
            acc_chunk_at((w + NW * i) * CHUNK)
            return carry

        lax.fori_loop(0, BASE_CH, acc_chunk, 0)

        @pl.when(w < EXTRA)
        def _():
            acc_chunk_at((w + NW * BASE_CH) * CHUNK)

        plsc.subcore_barrier()
        for j in range(NPT // ZROWS):
            rows = pl.ds(s * NPT + j * ZROWS, ZROWS)

            @pl.when(c == 0)
            def _():
                pltpu.sync_copy(acc.at[rows], out0_hbm.at[g, rows])

            @pl.when(c == 1)
            def _():
                pltpu.sync_copy(acc.at[rows], out1_hbm.at[g, rows])


@functools.lru_cache(maxsize=None)
def _make_scatter_k():
    return pl.kernel(
        _scatter_body,
        mesh=_mesh(),
        out_type=(
            jax.ShapeDtypeStruct((PG, NACC, PW), jnp.float32),
            jax.ShapeDtypeStruct((PG, NACC, PW), jnp.float32),
        ),
        scratch_types=[
            pltpu.VMEM((CHUNK,), jnp.int32),
            pltpu.VMEM((CHUNK, PW), jnp.float32),
            pltpu.VMEM((ZROWS, PW), jnp.float32),
            pltpu.VMEM_SHARED((NACC, PW), jnp.float32),
        ],
    )


TN = 200


def _tail_body(x0_ref, x1_ref, g_ref, b_ref, wm1_ref, bm1_ref, wm2_ref,
               bm2_ref, wIa_ref, wAa_ref, wSa_ref, o_ref):
    f32 = jnp.float32

    def comp(a, b):
        c = 3 * a + b
        g, off = c // 2, F * (c % 2)
        return (x0_ref[g, :, off:off + F] + x1_ref[g, :, off:off + F])

    X00, X01, X02 = comp(0, 0), comp(0, 1), comp(0, 2)
    X10, X11, X12 = comp(1, 0), comp(1, 1), comp(1, 2)
    X20, X21, X22 = comp(2, 0), comp(2, 1), comp(2, 2)
    norm = (X00 * X00 + X01 * X01 + X02 * X02
            + X10 * X10 + X11 * X11 + X12 * X12
            + X20 * X20 + X21 * X21 + X22 * X22)
    mu = jnp.mean(norm, axis=-1, keepdims=True)
    dv = norm - mu
    var = jnp.mean(dv * dv, axis=-1, keepdims=True)
    h = dv * lax.rsqrt(var + 1e-5) * g_ref[...] + b_ref[...]
    h1 = jnp.dot(h, wm1_ref[...], preferred_element_type=f32) + bm1_ref[...]
    h1 = h1 * (1.0 / (1.0 + jnp.exp(-h1)))
    fs = jnp.dot(h1, wm2_ref[...], preferred_element_type=f32) + bm2_ref[...]
    fs = fs * (1.0 / (1.0 + jnp.exp(-fs)))
    fI = fs[:, 0 * F:1 * F]
    fA = fs[:, 1 * F:2 * F]
    fS = fs[:, 2 * F:3 * F]
    tr3 = (X00 + X11 + X22) * (1.0 / 3.0)
    wIa = wIa_ref[...]
    wAa = wAa_ref[...]
    wSa = wSa_ref[...]
    t1 = jnp.dot(tr3, wIa, preferred_element_type=f32)
    a01 = jnp.dot(0.5 * (X01 - X10), wAa, preferred_element_type=f32)
    a02 = jnp.dot(0.5 * (X02 - X20), wAa, preferred_element_type=f32)
    a12 = jnp.dot(0.5 * (X12 - X21), wAa, preferred_element_type=f32)
    s00 = jnp.dot(X00 - tr3, wSa, preferred_element_type=f32)
    s01 = jnp.dot(0.5 * (X01 + X10), wSa, preferred_element_type=f32)
    s02 = jnp.dot(0.5 * (X02 + X20), wSa, preferred_element_type=f32)
    s11 = jnp.dot(X11 - tr3, wSa, preferred_element_type=f32)
    s12 = jnp.dot(0.5 * (X12 + X21), wSa, preferred_element_type=f32)
    s22 = -(s00 + s11)
    o_ref[:, 0, :] = fI * t1 + fS * s00
    o_ref[:, 1, :] = fA * a01 + fS * s01
    o_ref[:, 2, :] = fA * a02 + fS * s02
    o_ref[:, 3, :] = -(fA * a01) + fS * s01
    o_ref[:, 4, :] = fI * t1 + fS * s11
    o_ref[:, 5, :] = fA * a12 + fS * s12
    o_ref[:, 6, :] = -(fA * a02) + fS * s02
    o_ref[:, 7, :] = -(fA * a12) + fS * s12
    o_ref[:, 8, :] = fI * t1 + fS * s22


def _tail(x0, x1, ln_g, ln_b, wm1, bm1, wm2p, bm2p, wIa, wAa, wSa):
    full = lambda shape: pl.BlockSpec(shape, lambda i: tuple(0 for _ in shape))
    return pl.pallas_call(
        _tail_body,
        grid=(N // TN,),
        in_specs=[
            pl.BlockSpec((PG, TN, PW), lambda i: (0, i, 0)),
            pl.BlockSpec((PG, TN, PW), lambda i: (0, i, 0)),
            full((1, F)), full((1, F)), full((F, F)), full((1, F)),
            full((F, 3 * F)), full((1, 3 * F)),
            full((F, F)), full((F, F)), full((F, F)),
        ],
        out_specs=pl.BlockSpec((TN, 9, F), lambda i: (i, 0, 0)),
        out_shape=jax.ShapeDtypeStruct((N, 9, F), jnp.float32),
    )(x0, x1, ln_g, ln_b, wm1, bm1, wm2p, bm2p, wIa, wAa, wSa)


def kernel(coordinates, node_attrs, edge_attrs, edge_index, num_nodes, W_node,
           W_edge, W_cat, b_cat, W_I, b_I, W_A, b_A, W_S, b_S, ln_g, ln_b,
           W_m1, b_m1, W_m2, b_m2, W_Ia, W_Aa, W_Sa):
    del num_nodes
    nf = _node_feats(node_attrs, W_node, coordinates)
    eidx = edge_index.astype(jnp.int32)
    gj, gi = _make_gather_k()(nf, eidx)

    wcj = W_cat[0:F]
    wci = W_cat[F:2 * F]
    wce = W_cat[2 * F:3 * F]
    mu_row = jnp.linspace(float(np.exp(-CUT)), 1.0, R,
                          dtype=jnp.float32).reshape(1, R)
    p = _edge_stage(gj, gi, edge_attrs, wcj, wci, W_edge, wce,
                    b_cat.reshape(1, F), W_I, b_I.reshape(1, F), W_A,
                    b_A.reshape(1, F), W_S, b_S.reshape(1, F), mu_row)

    x0, x1 = _make_scatter_k()(eidx, p)

    wm2p = W_m2.reshape(F, F, 3).transpose(0, 2, 1).reshape(F, 3 * F)
    bm2p = b_m2.reshape(F, 3).T.reshape(1, 3 * F)
    out = _tail(x0, x1, ln_g.reshape(1, F), ln_b.reshape(1, F), W_m1,
                b_m1.reshape(1, F), wm2p, bm2p, W_Ia, W_Aa, W_Sa)
    return out.transpose(0, 2, 1).reshape(N, F, 3, 3)


# speedup vs baseline: 18.2122x; 1.2003x over previous
import functools

import jax
import jax.numpy as jnp
import numpy as np
from jax import lax
from jax.experimental import pallas as pl
from jax.experimental.pallas import tpu as pltpu
from jax.experimental.pallas import tpu_sc as plsc

N = 10000
E = 160000
NODE_F = 128
EDGE_F = 16
F = 64
R = 32
CUT = 5.0
BETA = (2.0 / R * (1.0 - float(np.exp(-CUT)))) ** (-2)

NC = 2
NS = 16
NW = NC * NS
CHUNK = 128
NCHUNKS = E // CHUNK
BASE_CH = NCHUNKS // NW
EXTRA = NCHUNKS - BASE_CH * NW
NACC = 10240
NPT = NACC // NS
ZROWS = 64

GW = 128
PG = 5
PW = 128


def _mesh():
    return plsc.VectorSubcoreMesh(core_axis_name="c", subcore_axis_name="s")


def _nf_body(na_ref, w_ref, c_ref, o_ref):
    o_ref[:, 0:F] = jnp.dot(na_ref[...], w_ref[...],
                            preferred_element_type=jnp.float32)
    o_ref[:, F:F + 3] = c_ref[...]
    o_ref[:, F + 3:GW] = jnp.zeros((N, GW - F - 3), jnp.float32)


def _node_feats(node_attrs, w_node, coordinates):
    return pl.pallas_call(
        _nf_body,
        out_shape=jax.ShapeDtypeStruct((N, GW), jnp.float32),
    )(node_attrs, w_node, coordinates)


def _gather_body(nf_hbm, eidx_hbm, gj_hbm, gi_hbm,
                 sb0, sb1, rb0, rb1, jr0, jr1, ir0, ir1,
                 smj0, smj1, smi0, smi1):
    c = lax.axis_index("c")
    s = lax.axis_index("s")
    w = c * NS + s
    sbufs, rbufs = (sb0, sb1), (rb0, rb1)
    jrows, irows = (jr0, jr1), (ir0, ir1)
    jsems, isems = (smj0, smj1), (smi0, smi1)

    def load_idx(i, sl):
        base = (w + NW * i) * CHUNK
        pltpu.sync_copy(eidx_hbm.at[0, pl.ds(base, CHUNK)], sbufs[sl])
        pltpu.sync_copy(eidx_hbm.at[1, pl.ds(base, CHUNK)], rbufs[sl])
        pltpu.async_copy(nf_hbm.at[rbufs[sl]], jrows[sl], jsems[sl])
        pltpu.async_copy(nf_hbm.at[sbufs[sl]], irows[sl], isems[sl])

    def fin(i, sl):
        base = (w + NW * i) * CHUNK
        pltpu.make_async_copy(nf_hbm.at[pl.ds(0, CHUNK)], jrows[sl],
                              jsems[sl]).wait()
        pltpu.make_async_copy(nf_hbm.at[pl.ds(0, CHUNK)], irows[sl],
                              isems[sl]).wait()
        pltpu.sync_copy(jrows[sl], gj_hbm.at[pl.ds(base, CHUNK)])
        pltpu.sync_copy(irows[sl], gi_hbm.at[pl.ds(base, CHUNK)])

    load_idx(0, 0)
    load_idx(1, 1)

    def pair(i2, carry):
        a = 2 * i2
        fin(a, 0)
        load_idx(a + 2, 0)
        fin(a + 1, 1)

        @pl.when(a + 3 < BASE_CH)
        def _():
            load_idx(a + 3, 1)

        return carry

    lax.fori_loop(0, (BASE_CH - 1) // 2, pair, 0)
    fin(BASE_CH - 1, 0)

    @pl.when(w < EXTRA)
    def _():
        load_idx(BASE_CH, 0)
        fin(BASE_CH, 0)


@functools.lru_cache(maxsize=None)
def _make_gather_k():
    return pl.kernel(
        _gather_body,
        mesh=_mesh(),
        out_type=(
            jax.ShapeDtypeStruct((E, GW), jnp.float32),
            jax.ShapeDtypeStruct((E, GW), jnp.float32),
        ),
        scratch_types=[
            pltpu.VMEM((CHUNK,), jnp.int32),
            pltpu.VMEM((CHUNK,), jnp.int32),
            pltpu.VMEM((CHUNK,), jnp.int32),
            pltpu.VMEM((CHUNK,), jnp.int32),
            pltpu.VMEM((CHUNK, GW), jnp.float32),
            pltpu.VMEM((CHUNK, GW), jnp.float32),
            pltpu.VMEM((CHUNK, GW), jnp.float32),
            pltpu.VMEM((CHUNK, GW), jnp.float32),
            pltpu.SemaphoreType.DMA,
            pltpu.SemaphoreType.DMA,
            pltpu.SemaphoreType.DMA,
            pltpu.SemaphoreType.DMA,
        ],
    )


TE = 256


def _edge_body(gj_ref, gi_ref, ea_ref, wcj_ref, wci_ref, we_ref,
               wce_ref, bcat_ref, wI_ref, bI_ref, wA_ref, bA_ref, wS_ref,
               bS_ref, mu_ref, p_ref):
    f32 = jnp.float32
    ef = jnp.dot(ea_ref[...], we_ref[...], preferred_element_type=f32)
    nfji = (jnp.dot(gj_ref[:, 0:F], wcj_ref[...], preferred_element_type=f32)
            + jnp.dot(gi_ref[:, 0:F], wci_ref[...], preferred_element_type=f32)
            + jnp.dot(ef, wce_ref[...], preferred_element_type=f32)
            + bcat_ref[...])
    vx = gi_ref[:, F:F + 1] - gj_ref[:, F:F + 1]
    vy = gi_ref[:, F + 1:F + 2] - gj_ref[:, F + 1:F + 2]
    vz = gi_ref[:, F + 2:F + 3] - gj_ref[:, F + 2:F + 3]
    lsq = vx * vx + vy * vy + vz * vz
    l = jnp.sqrt(lsq)
    inv = 1.0 / l
    nx = vx * inv
    ny = vy * inv
    nz = vz * inv
    d = jnp.exp(-l) - mu_ref[...]
    rbf = jnp.exp((-BETA) * d * d)
    phi = jnp.where(l < CUT, 0.5 * (jnp.cos((np.pi / CUT) * l) + 1.0), 0.0)
    q = rbf * phi
    f0I = (jnp.dot(q, wI_ref[...], preferred_element_type=f32) + bI_ref[...]) * phi
    f0A = (jnp.dot(q, wA_ref[...], preferred_element_type=f32) + bA_ref[...]) * phi
    f0S = (jnp.dot(q, wS_ref[...], preferred_element_type=f32) + bS_ref[...]) * phi
    gI = nfji * f0I
    gA = nfji * f0A
    gS = nfji * f0S
    t = 1.0 / 3.0
    m = [
        gI + gS * (nx * nx - t),
        gS * (nx * ny) - gA * nz,
        gS * (nx * nz) + gA * ny,
        gS * (nx * ny) + gA * nz,
        gI + gS * (ny * ny - t),
        gS * (ny * nz) - gA * nx,
        gS * (nx * nz) - gA * ny,
        gS * (ny * nz) + gA * nx,
        gI + gS * (nz * nz - t),
    ]
    for g in range(4):
        p_ref[g, :, 0:F] = m[2 * g]
        p_ref[g, :, F:PW] = m[2 * g + 1]
    p_ref[4, :, 0:F] = m[8]
    p_ref[4, :, F:PW] = jnp.zeros_like(m[8])


def _edge_stage(gj, gi, ea, wcj, wci, w_edge, wce, bcat, wI, bI, wA,
                bA, wS, bS, mu_row):
    n_blocks = E // TE
    full = lambda shape: pl.BlockSpec(shape, lambda i: tuple(0 for _ in shape))
    return pl.pallas_call(
        _edge_body,
        grid=(n_blocks,),
        in_specs=[
            pl.BlockSpec((TE, GW), lambda i: (i, 0)),
            pl.BlockSpec((TE, GW), lambda i: (i, 0)),
            pl.BlockSpec((TE, EDGE_F), lambda i: (i, 0)),
            full((F, F)), full((F, F)), full((EDGE_F, F)), full((F, F)),
            full((1, F)), full((R, F)), full((1, F)), full((R, F)),
            full((1, F)), full((R, F)), full((1, F)), full((1, R)),
        ],
        out_specs=pl.BlockSpec((PG, TE, PW), lambda i: (0, i, 0)),
        out_shape=jax.ShapeDtypeStruct((PG, E, PW), jnp.float32),
    )(gj, gi, ea, wcj, wci, w_edge, wce, bcat, wI, bI, wA, bA, wS, bS,
      mu_row)


def _scatter_body(eidx_hbm, p_hbm, out0_hbm, out1_hbm,
                  ib0, ib1, ub0, ub1, zbuf, acc, si0, si1, su0, su1):
    c = lax.axis_index("c")
    s = lax.axis_index("s")
    w = c * NS + s
    ibufs, ubufs = (ib0, ib1), (ub0, ub1)
    isems, usems = (si0, si1), (su0, su1)
    zv = jnp.zeros((16,), jnp.float32)

    def zrow(r, carry):
        def zcol(j, carry2):
            zbuf[r, pl.ds(j * 16, 16)] = zv
            return carry2
        return lax.fori_loop(0, PW // 16, zcol, carry)

    lax.fori_loop(0, ZROWS, zrow, 0)

    for g in range(PG):
        plsc.subcore_barrier()
        for j in range(NPT // ZROWS):
            pltpu.sync_copy(zbuf, acc.at[pl.ds(s * NPT + j * ZROWS, ZROWS)])
        plsc.subcore_barrier()

        def start(i, sl):
            base = (w + NW * i) * CHUNK
            pltpu.async_copy(eidx_hbm.at[1, pl.ds(base, CHUNK)], ibufs[sl],
                             isems[sl])
            pltpu.async_copy(p_hbm.at[g, pl.ds(base, CHUNK)], ubufs[sl],
                             usems[sl])

        def wait_scatter(sl):
            pltpu.make_async_copy(eidx_hbm.at[1, pl.ds(0, CHUNK)], ibufs[sl],
                                  isems[sl]).wait()
            pltpu.make_async_copy(p_hbm.at[g, pl.ds(0, CHUNK)], ubufs[sl],
                                  usems[sl]).wait()
            pltpu.sync_copy(ubufs[sl], acc.at[ibufs[sl]], add=True)

        start(0, 0)

        def pair(i2, carry):
            a = 2 * i2
            start(a + 1, 1)
            wait_scatter(0)
            start(a + 2, 0)
            wait_scatter(1)
            return carry

        lax.fori_loop(0, (BASE_CH - 1) // 2, pair, 0)
        wait_scatter(0)

        @pl.when(w < EXTRA)
        def _():
            start(BASE_CH, 1)
            wait_scatter(1)

        plsc.subcore_barrier()
        for j in range(NPT // ZROWS):
            rows = pl.ds(s * NPT + j * ZROWS, ZROWS)

            @pl.when(c == 0)
            def _():
                pltpu.sync_copy(acc.at[rows], out0_hbm.at[g, rows])

            @pl.when(c == 1)
            def _():
                pltpu.sync_copy(acc.at[rows], out1_hbm.at[g, rows])


@functools.lru_cache(maxsize=None)
def _make_scatter_k():
    return pl.kernel(
        _scatter_body,
        mesh=_mesh(),
        out_type=(
            jax.ShapeDtypeStruct((PG, NACC, PW), jnp.float32),
            jax.ShapeDtypeStruct((PG, NACC, PW), jnp.float32),
        ),
        scratch_types=[
            pltpu.VMEM((CHUNK,), jnp.int32),
            pltpu.VMEM((CHUNK,), jnp.int32),
            pltpu.VMEM((CHUNK, PW), jnp.float32),
            pltpu.VMEM((CHUNK, PW), jnp.float32),
            pltpu.VMEM((ZROWS, PW), jnp.float32),
            pltpu.VMEM_SHARED((NACC, PW), jnp.float32),
            pltpu.SemaphoreType.DMA,
            pltpu.SemaphoreType.DMA,
            pltpu.SemaphoreType.DMA,
            pltpu.SemaphoreType.DMA,
        ],
    )


TN = 200


def _tail_body(x0_ref, x1_ref, g_ref, b_ref, wm1_ref, bm1_ref, wm2_ref,
               bm2_ref, wIa_ref, wAa_ref, wSa_ref, o_ref):
    f32 = jnp.float32

    def comp(a, b):
        cc = 3 * a + b
        g, off = cc // 2, F * (cc % 2)
        return (x0_ref[g, :, off:off + F].astype(f32)
                + x1_ref[g, :, off:off + F].astype(f32))

    X00, X01, X02 = comp(0, 0), comp(0, 1), comp(0, 2)
    X10, X11, X12 = comp(1, 0), comp(1, 1), comp(1, 2)
    X20, X21, X22 = comp(2, 0), comp(2, 1), comp(2, 2)
    norm = (X00 * X00 + X01 * X01 + X02 * X02
            + X10 * X10 + X11 * X11 + X12 * X12
            + X20 * X20 + X21 * X21 + X22 * X22)
    mu = jnp.mean(norm, axis=-1, keepdims=True)
    dv = norm - mu
    var = jnp.mean(dv * dv, axis=-1, keepdims=True)
    h = dv * lax.rsqrt(var + 1e-5) * g_ref[...] + b_ref[...]
    h1 = jnp.dot(h, wm1_ref[...], preferred_element_type=f32) + bm1_ref[...]
    h1 = h1 * (1.0 / (1.0 + jnp.exp(-h1)))
    fs = jnp.dot(h1, wm2_ref[...], preferred_element_type=f32) + bm2_ref[...]
    fs = fs * (1.0 / (1.0 + jnp.exp(-fs)))
    fI = fs[:, 0 * F:1 * F]
    fA = fs[:, 1 * F:2 * F]
    fS = fs[:, 2 * F:3 * F]
    tr3 = (X00 + X11 + X22) * (1.0 / 3.0)
    wIa = wIa_ref[...]
    wAa = wAa_ref[...]
    wSa = wSa_ref[...]
    t1 = jnp.dot(tr3, wIa, preferred_element_type=f32)
    a01 = jnp.dot(0.5 * (X01 - X10), wAa, preferred_element_type=f32)
    a02 = jnp.dot(0.5 * (X02 - X20), wAa, preferred_element_type=f32)
    a12 = jnp.dot(0.5 * (X12 - X21), wAa, preferred_element_type=f32)
    s00 = jnp.dot(X00 - tr3, wSa, preferred_element_type=f32)
    s01 = jnp.dot(0.5 * (X01 + X10), wSa, preferred_element_type=f32)
    s02 = jnp.dot(0.5 * (X02 + X20), wSa, preferred_element_type=f32)
    s11 = jnp.dot(X11 - tr3, wSa, preferred_element_type=f32)
    s12 = jnp.dot(0.5 * (X12 + X21), wSa, preferred_element_type=f32)
    s22 = -(s00 + s11)
    o_ref[:, 0, :] = fI * t1 + fS * s00
    o_ref[:, 1, :] = fA * a01 + fS * s01
    o_ref[:, 2, :] = fA * a02 + fS * s02
    o_ref[:, 3, :] = -(fA * a01) + fS * s01
    o_ref[:, 4, :] = fI * t1 + fS * s11
    o_ref[:, 5, :] = fA * a12 + fS * s12
    o_ref[:, 6, :] = -(fA * a02) + fS * s02
    o_ref[:, 7, :] = -(fA * a12) + fS * s12
    o_ref[:, 8, :] = fI * t1 + fS * s22


def _tail(x0, x1, ln_g, ln_b, wm1, bm1, wm2p, bm2p, wIa, wAa, wSa):
    full = lambda shape: pl.BlockSpec(shape, lambda i: tuple(0 for _ in shape))
    return pl.pallas_call(
        _tail_body,
        grid=(N // TN,),
        in_specs=[
            pl.BlockSpec((PG, TN, PW), lambda i: (0, i, 0)),
            pl.BlockSpec((PG, TN, PW), lambda i: (0, i, 0)),
            full((1, F)), full((1, F)), full((F, F)), full((1, F)),
            full((F, 3 * F)), full((1, 3 * F)),
            full((F, F)), full((F, F)), full((F, F)),
        ],
        out_specs=pl.BlockSpec((TN, 9, F), lambda i: (i, 0, 0)),
        out_shape=jax.ShapeDtypeStruct((N, 9, F), jnp.float32),
    )(x0, x1, ln_g, ln_b, wm1, bm1, wm2p, bm2p, wIa, wAa, wSa)


def kernel(coordinates, node_attrs, edge_attrs, edge_index, num_nodes, W_node,
           W_edge, W_cat, b_cat, W_I, b_I, W_A, b_A, W_S, b_S, ln_g, ln_b,
           W_m1, b_m1, W_m2, b_m2, W_Ia, W_Aa, W_Sa):
    del num_nodes
    nf = _node_feats(node_attrs, W_node, coordinates)
    eidx = edge_index.astype(jnp.int32)
    gj, gi = _make_gather_k()(nf, eidx)

    wcj = W_cat[0:F]
    wci = W_cat[F:2 * F]
    wce = W_cat[2 * F:3 * F]
    mu_row = jnp.linspace(float(np.exp(-CUT)), 1.0, R,
                          dtype=jnp.float32).reshape(1, R)
    p = _edge_stage(gj, gi, edge_attrs, wcj, wci, W_edge, wce,
                    b_cat.reshape(1, F), W_I, b_I.reshape(1, F), W_A,
                    b_A.reshape(1, F), W_S, b_S.reshape(1, F), mu_row)

    x0, x1 = _make_scatter_k()(eidx, p)

    wm2p = W_m2.reshape(F, F, 3).transpose(0, 2, 1).reshape(F, 3 * F)
    bm2p = b_m2.reshape(F, 3).T.reshape(1, 3 * F)
    out = _tail(x0, x1, ln_g.reshape(1, F), ln_b.reshape(1, F), W_m1,
                b_m1.reshape(1, F), wm2p, bm2p, W_Ia, W_Aa, W_Sa)
    return out.transpose(0, 2, 1).reshape(N, F, 3, 3)


# speedup vs baseline: 20.3550x; 1.1177x over previous
import functools

import jax
import jax.numpy as jnp
import numpy as np
from jax import lax
from jax.experimental import pallas as pl
from jax.experimental.pallas import tpu as pltpu
from jax.experimental.pallas import tpu_sc as plsc

N = 10000
E = 160000
NODE_F = 128
EDGE_F = 16
F = 64
R = 32
CUT = 5.0
BETA = (2.0 / R * (1.0 - float(np.exp(-CUT)))) ** (-2)

NC = 2
NS = 16
NW = NC * NS
CHUNK = 128
NCHUNKS = E // CHUNK
BASE_CH = NCHUNKS // NW
EXTRA = NCHUNKS - BASE_CH * NW
NACC = 10240
NPT = NACC // NS
ZROWS = 64

GW = 128
PG = 5
PW = 128


def _mesh():
    return plsc.VectorSubcoreMesh(core_axis_name="c", subcore_axis_name="s")


def _nf_body(na_ref, w_ref, c_ref, o_ref):
    o_ref[:, 0:F] = jnp.dot(na_ref[...], w_ref[...],
                            preferred_element_type=jnp.float32)
    o_ref[:, F:F + 3] = c_ref[...]
    o_ref[:, F + 3:GW] = jnp.zeros((N, GW - F - 3), jnp.float32)


def _node_feats(node_attrs, w_node, coordinates):
    return pl.pallas_call(
        _nf_body,
        out_shape=jax.ShapeDtypeStruct((N, GW), jnp.float32),
    )(node_attrs, w_node, coordinates)


def _gather_body(nf_hbm, eidx_hbm, gj_hbm, gi_hbm,
                 sb0, sb1, rb0, rb1, jr0, jr1, ir0, ir1,
                 smj0, smj1, smi0, smi1):
    c = lax.axis_index("c")
    s = lax.axis_index("s")
    w = c * NS + s
    sbufs, rbufs = (sb0, sb1), (rb0, rb1)
    jrows, irows = (jr0, jr1), (ir0, ir1)
    jsems, isems = (smj0, smj1), (smi0, smi1)

    def load_idx(i, sl):
        base = (w + NW * i) * CHUNK
        pltpu.sync_copy(eidx_hbm.at[0, pl.ds(base, CHUNK)], sbufs[sl])
        pltpu.sync_copy(eidx_hbm.at[1, pl.ds(base, CHUNK)], rbufs[sl])
        pltpu.async_copy(nf_hbm.at[rbufs[sl]], jrows[sl], jsems[sl])
        pltpu.async_copy(nf_hbm.at[sbufs[sl]], irows[sl], isems[sl])

    def fin(i, sl):
        base = (w + NW * i) * CHUNK
        pltpu.make_async_copy(nf_hbm.at[pl.ds(0, CHUNK)], jrows[sl],
                              jsems[sl]).wait()
        pltpu.make_async_copy(nf_hbm.at[pl.ds(0, CHUNK)], irows[sl],
                              isems[sl]).wait()
        pltpu.sync_copy(jrows[sl], gj_hbm.at[pl.ds(base, CHUNK)])
        pltpu.sync_copy(irows[sl], gi_hbm.at[pl.ds(base, CHUNK)])

    load_idx(0, 0)
    load_idx(1, 1)

    def pair(i2, carry):
        a = 2 * i2
        fin(a, 0)
        load_idx(a + 2, 0)
        fin(a + 1, 1)

        @pl.when(a + 3 < BASE_CH)
        def _():
            load_idx(a + 3, 1)

        return carry

    lax.fori_loop(0, (BASE_CH - 1) // 2, pair, 0)
    fin(BASE_CH - 1, 0)

    @pl.when(w < EXTRA)
    def _():
        load_idx(BASE_CH, 0)
        fin(BASE_CH, 0)


@functools.lru_cache(maxsize=None)
def _make_gather_k():
    return pl.kernel(
        _gather_body,
        mesh=_mesh(),
        out_type=(
            jax.ShapeDtypeStruct((E, GW), jnp.float32),
            jax.ShapeDtypeStruct((E, GW), jnp.float32),
        ),
        scratch_types=[
            pltpu.VMEM((CHUNK,), jnp.int32),
            pltpu.VMEM((CHUNK,), jnp.int32),
            pltpu.VMEM((CHUNK,), jnp.int32),
            pltpu.VMEM((CHUNK,), jnp.int32),
            pltpu.VMEM((CHUNK, GW), jnp.float32),
            pltpu.VMEM((CHUNK, GW), jnp.float32),
            pltpu.VMEM((CHUNK, GW), jnp.float32),
            pltpu.VMEM((CHUNK, GW), jnp.float32),
            pltpu.SemaphoreType.DMA,
            pltpu.SemaphoreType.DMA,
            pltpu.SemaphoreType.DMA,
            pltpu.SemaphoreType.DMA,
        ],
    )


TE = 640


def _edge_body(gj_ref, gi_ref, ea_ref, wcj_ref, wci_ref, we_ref,
               wce_ref, bcat_ref, wI_ref, bI_ref, wA_ref, bA_ref, wS_ref,
               bS_ref, mu_ref, p_ref):
    f32 = jnp.float32
    ef = jnp.dot(ea_ref[...], we_ref[...], preferred_element_type=f32)
    nfji = (jnp.dot(gj_ref[:, 0:F], wcj_ref[...], preferred_element_type=f32)
            + jnp.dot(gi_ref[:, 0:F], wci_ref[...], preferred_element_type=f32)
            + jnp.dot(ef, wce_ref[...], preferred_element_type=f32)
            + bcat_ref[...])
    vx = gi_ref[:, F:F + 1] - gj_ref[:, F:F + 1]
    vy = gi_ref[:, F + 1:F + 2] - gj_ref[:, F + 1:F + 2]
    vz = gi_ref[:, F + 2:F + 3] - gj_ref[:, F + 2:F + 3]
    lsq = vx * vx + vy * vy + vz * vz
    l = jnp.sqrt(lsq)
    inv = 1.0 / l
    nx = vx * inv
    ny = vy * inv
    nz = vz * inv
    d = jnp.exp(-l) - mu_ref[...]
    rbf = jnp.exp((-BETA) * d * d)
    phi = jnp.where(l < CUT, 0.5 * (jnp.cos((np.pi / CUT) * l) + 1.0), 0.0)
    q = rbf * phi
    f0I = (jnp.dot(q, wI_ref[...], preferred_element_type=f32) + bI_ref[...]) * phi
    f0A = (jnp.dot(q, wA_ref[...], preferred_element_type=f32) + bA_ref[...]) * phi
    f0S = (jnp.dot(q, wS_ref[...], preferred_element_type=f32) + bS_ref[...]) * phi
    gI = nfji * f0I
    gA = nfji * f0A
    gS = nfji * f0S
    t = 1.0 / 3.0
    m = [
        gI + gS * (nx * nx - t),
        gS * (nx * ny) - gA * nz,
        gS * (nx * nz) + gA * ny,
        gS * (nx * ny) + gA * nz,
        gI + gS * (ny * ny - t),
        gS * (ny * nz) - gA * nx,
        gS * (nx * nz) - gA * ny,
        gS * (ny * nz) + gA * nx,
        gI + gS * (nz * nz - t),
    ]
    for g in range(4):
        p_ref[g, :, 0:F] = m[2 * g]
        p_ref[g, :, F:PW] = m[2 * g + 1]
    p_ref[4, :, 0:F] = m[8]
    p_ref[4, :, F:PW] = jnp.zeros_like(m[8])


def _edge_stage(gj, gi, ea, wcj, wci, w_edge, wce, bcat, wI, bI, wA,
                bA, wS, bS, mu_row):
    n_blocks = E // TE
    full = lambda shape: pl.BlockSpec(shape, lambda i: tuple(0 for _ in shape))
    return pl.pallas_call(
        _edge_body,
        grid=(n_blocks,),
        in_specs=[
            pl.BlockSpec((TE, GW), lambda i: (i, 0)),
            pl.BlockSpec((TE, GW), lambda i: (i, 0)),
            pl.BlockSpec((TE, EDGE_F), lambda i: (i, 0)),
            full((F, F)), full((F, F)), full((EDGE_F, F)), full((F, F)),
            full((1, F)), full((R, F)), full((1, F)), full((R, F)),
            full((1, F)), full((R, F)), full((1, F)), full((1, R)),
        ],
        out_specs=pl.BlockSpec((PG, TE, PW), lambda i: (0, i, 0)),
        out_shape=jax.ShapeDtypeStruct((PG, E, PW), jnp.float32),
    )(gj, gi, ea, wcj, wci, w_edge, wce, bcat, wI, bI, wA, bA, wS, bS,
      mu_row)


def _scatter_body(eidx_hbm, p_hbm, out0_hbm, out1_hbm,
                  ib0, ib1, ub0, ub1, zbuf, acc, si0, si1, su0, su1):
    c = lax.axis_index("c")
    s = lax.axis_index("s")
    w = c * NS + s
    ibufs, ubufs = (ib0, ib1), (ub0, ub1)
    isems, usems = (si0, si1), (su0, su1)
    zv = jnp.zeros((16,), jnp.float32)

    def zrow(r, carry):
        def zcol(j, carry2):
            zbuf[r, pl.ds(j * 16, 16)] = zv
            return carry2
        return lax.fori_loop(0, PW // 16, zcol, carry)

    lax.fori_loop(0, ZROWS, zrow, 0)

    for g in range(PG):
        plsc.subcore_barrier()
        for j in range(NPT // ZROWS):
            pltpu.sync_copy(zbuf, acc.at[pl.ds(s * NPT + j * ZROWS, ZROWS)])
        plsc.subcore_barrier()

        def start(i, sl):
            base = (w + NW * i) * CHUNK
            pltpu.async_copy(eidx_hbm.at[1, pl.ds(base, CHUNK)], ibufs[sl],
                             isems[sl])
            pltpu.async_copy(p_hbm.at[g, pl.ds(base, CHUNK)], ubufs[sl],
                             usems[sl])

        def wait_scatter(sl):
            pltpu.make_async_copy(eidx_hbm.at[1, pl.ds(0, CHUNK)], ibufs[sl],
                                  isems[sl]).wait()
            pltpu.make_async_copy(p_hbm.at[g, pl.ds(0, CHUNK)], ubufs[sl],
                                  usems[sl]).wait()
            pltpu.sync_copy(ubufs[sl], acc.at[ibufs[sl]], add=True)

        start(0, 0)

        def pair(i2, carry):
            a = 2 * i2
            start(a + 1, 1)
            wait_scatter(0)
            start(a + 2, 0)
            wait_scatter(1)
            return carry

        lax.fori_loop(0, (BASE_CH - 1) // 2, pair, 0)
        wait_scatter(0)

        @pl.when(w < EXTRA)
        def _():
            start(BASE_CH, 1)
            wait_scatter(1)

        plsc.subcore_barrier()
        for j in range(NPT // ZROWS):
            rows = pl.ds(s * NPT + j * ZROWS, ZROWS)

            @pl.when(c == 0)
            def _():
                pltpu.sync_copy(acc.at[rows], out0_hbm.at[g, rows])

            @pl.when(c == 1)
            def _():
                pltpu.sync_copy(acc.at[rows], out1_hbm.at[g, rows])


@functools.lru_cache(maxsize=None)
def _make_scatter_k():
    return pl.kernel(
        _scatter_body,
        mesh=_mesh(),
        out_type=(
            jax.ShapeDtypeStruct((PG, NACC, PW), jnp.float32),
            jax.ShapeDtypeStruct((PG, NACC, PW), jnp.float32),
        ),
        scratch_types=[
            pltpu.VMEM((CHUNK,), jnp.int32),
            pltpu.VMEM((CHUNK,), jnp.int32),
            pltpu.VMEM((CHUNK, PW), jnp.float32),
            pltpu.VMEM((CHUNK, PW), jnp.float32),
            pltpu.VMEM((ZROWS, PW), jnp.float32),
            pltpu.VMEM_SHARED((NACC, PW), jnp.float32),
            pltpu.SemaphoreType.DMA,
            pltpu.SemaphoreType.DMA,
            pltpu.SemaphoreType.DMA,
            pltpu.SemaphoreType.DMA,
        ],
    )


TN = 400


def _tail_body(x0_ref, x1_ref, g_ref, b_ref, wm1_ref, bm1_ref, wm2_ref,
               bm2_ref, wIa_ref, wAa_ref, wSa_ref, o_ref):
    f32 = jnp.float32

    def comp(a, b):
        cc = 3 * a + b
        g, off = cc // 2, F * (cc % 2)
        return (x0_ref[g, :, off:off + F].astype(f32)
                + x1_ref[g, :, off:off + F].astype(f32))

    X00, X01, X02 = comp(0, 0), comp(0, 1), comp(0, 2)
    X10, X11, X12 = comp(1, 0), comp(1, 1), comp(1, 2)
    X20, X21, X22 = comp(2, 0), comp(2, 1), comp(2, 2)
    norm = (X00 * X00 + X01 * X01 + X02 * X02
            + X10 * X10 + X11 * X11 + X12 * X12
            + X20 * X20 + X21 * X21 + X22 * X22)
    mu = jnp.mean(norm, axis=-1, keepdims=True)
    dv = norm - mu
    var = jnp.mean(dv * dv, axis=-1, keepdims=True)
    h = dv * lax.rsqrt(var + 1e-5) * g_ref[...] + b_ref[...]
    h1 = jnp.dot(h, wm1_ref[...], preferred_element_type=f32) + bm1_ref[...]
    h1 = h1 * (1.0 / (1.0 + jnp.exp(-h1)))
    fs = jnp.dot(h1, wm2_ref[...], preferred_element_type=f32) + bm2_ref[...]
    fs = fs * (1.0 / (1.0 + jnp.exp(-fs)))
    fI = fs[:, 0 * F:1 * F]
    fA = fs[:, 1 * F:2 * F]
    fS = fs[:, 2 * F:3 * F]
    tr3 = (X00 + X11 + X22) * (1.0 / 3.0)
    wIa = wIa_ref[...]
    wAa = wAa_ref[...]
    wSa = wSa_ref[...]
    t1 = jnp.dot(tr3, wIa, preferred_element_type=f32)
    a01 = jnp.dot(0.5 * (X01 - X10), wAa, preferred_element_type=f32)
    a02 = jnp.dot(0.5 * (X02 - X20), wAa, preferred_element_type=f32)
    a12 = jnp.dot(0.5 * (X12 - X21), wAa, preferred_element_type=f32)
    s00 = jnp.dot(X00 - tr3, wSa, preferred_element_type=f32)
    s01 = jnp.dot(0.5 * (X01 + X10), wSa, preferred_element_type=f32)
    s02 = jnp.dot(0.5 * (X02 + X20), wSa, preferred_element_type=f32)
    s11 = jnp.dot(X11 - tr3, wSa, preferred_element_type=f32)
    s12 = jnp.dot(0.5 * (X12 + X21), wSa, preferred_element_type=f32)
    s22 = -(s00 + s11)
    o_ref[:, 0, :] = fI * t1 + fS * s00
    o_ref[:, 1, :] = fA * a01 + fS * s01
    o_ref[:, 2, :] = fA * a02 + fS * s02
    o_ref[:, 3, :] = -(fA * a01) + fS * s01
    o_ref[:, 4, :] = fI * t1 + fS * s11
    o_ref[:, 5, :] = fA * a12 + fS * s12
    o_ref[:, 6, :] = -(fA * a02) + fS * s02
    o_ref[:, 7, :] = -(fA * a12) + fS * s12
    o_ref[:, 8, :] = fI * t1 + fS * s22


def _tail(x0, x1, ln_g, ln_b, wm1, bm1, wm2p, bm2p, wIa, wAa, wSa):
    full = lambda shape: pl.BlockSpec(shape, lambda i: tuple(0 for _ in shape))
    return pl.pallas_call(
        _tail_body,
        grid=(N // TN,),
        in_specs=[
            pl.BlockSpec((PG, TN, PW), lambda i: (0, i, 0)),
            pl.BlockSpec((PG, TN, PW), lambda i: (0, i, 0)),
            full((1, F)), full((1, F)), full((F, F)), full((1, F)),
            full((F, 3 * F)), full((1, 3 * F)),
            full((F, F)), full((F, F)), full((F, F)),
        ],
        out_specs=pl.BlockSpec((TN, 9, F), lambda i: (i, 0, 0)),
        out_shape=jax.ShapeDtypeStruct((N, 9, F), jnp.float32),
    )(x0, x1, ln_g, ln_b, wm1, bm1, wm2p, bm2p, wIa, wAa, wSa)


def kernel(coordinates, node_attrs, edge_attrs, edge_index, num_nodes, W_node,
           W_edge, W_cat, b_cat, W_I, b_I, W_A, b_A, W_S, b_S, ln_g, ln_b,
           W_m1, b_m1, W_m2, b_m2, W_Ia, W_Aa, W_Sa):
    del num_nodes
    nf = _node_feats(node_attrs, W_node, coordinates)
    eidx = edge_index.astype(jnp.int32)
    gj, gi = _make_gather_k()(nf, eidx)

    wcj = W_cat[0:F]
    wci = W_cat[F:2 * F]
    wce = W_cat[2 * F:3 * F]
    mu_row = jnp.linspace(float(np.exp(-CUT)), 1.0, R,
                          dtype=jnp.float32).reshape(1, R)
    p = _edge_stage(gj, gi, edge_attrs, wcj, wci, W_edge, wce,
                    b_cat.reshape(1, F), W_I, b_I.reshape(1, F), W_A,
                    b_A.reshape(1, F), W_S, b_S.reshape(1, F), mu_row)

    x0, x1 = _make_scatter_k()(eidx, p)

    wm2p = W_m2.reshape(F, F, 3).transpose(0, 2, 1).reshape(F, 3 * F)
    bm2p = b_m2.reshape(F, 3).T.reshape(1, 3 * F)
    out = _tail(x0, x1, ln_g.reshape(1, F), ln_b.reshape(1, F), W_m1,
                b_m1.reshape(1, F), wm2p, bm2p, W_Ia, W_Aa, W_Sa)
    return out.transpose(0, 2, 1).reshape(N, F, 3, 3)


# speedup vs baseline: 22.8326x; 1.1217x over previous
import functools

import jax
import jax.numpy as jnp
import numpy as np
from jax import lax
from jax.experimental import pallas as pl
from jax.experimental.pallas import tpu as pltpu
from jax.experimental.pallas import tpu_sc as plsc

N = 10000
E = 160000
NODE_F = 128
EDGE_F = 16
F = 64
R = 32
CUT = 5.0
BETA = (2.0 / R * (1.0 - float(np.exp(-CUT)))) ** (-2)

NC = 2
NS = 16
NW = NC * NS
CHUNK = 128
NCHUNKS = E // CHUNK
BASE_CH = NCHUNKS // NW
EXTRA = NCHUNKS - BASE_CH * NW
EH = E // 2
NCH_H = NCHUNKS // 2
BASE_H = NCH_H // NW
EXTRA_H = NCH_H - BASE_H * NW
NACC = 10240
NPT = NACC // NS
ZROWS = 64

GW = 128
PG = 5
PW = 128


def _mesh():
    return plsc.VectorSubcoreMesh(core_axis_name="c", subcore_axis_name="s")


def _nf_body(na_ref, w_ref, c_ref, o_ref):
    o_ref[:, 0:F] = jnp.dot(na_ref[...], w_ref[...],
                            preferred_element_type=jnp.float32)
    o_ref[:, F:F + 3] = c_ref[...]
    o_ref[:, F + 3:GW] = jnp.zeros((N, GW - F - 3), jnp.float32)


def _node_feats(node_attrs, w_node, coordinates):
    return pl.pallas_call(
        _nf_body,
        out_shape=jax.ShapeDtypeStruct((N, GW), jnp.float32),
    )(node_attrs, w_node, coordinates)


def _gather_body(half, nf_hbm, eidx_hbm, gj_hbm, gi_hbm,
                 sb0, sb1, rb0, rb1, jr0, jr1, ir0, ir1,
                 smj0, smj1, smi0, smi1):
    hbase = NCH_H * half
    c = lax.axis_index("c")
    s = lax.axis_index("s")
    w = c * NS + s
    sbufs, rbufs = (sb0, sb1), (rb0, rb1)
    jrows, irows = (jr0, jr1), (ir0, ir1)
    jsems, isems = (smj0, smj1), (smi0, smi1)

    def load_idx(i, sl):
        base = (hbase + w + NW * i) * CHUNK
        pltpu.sync_copy(eidx_hbm.at[0, pl.ds(base, CHUNK)], sbufs[sl])
        pltpu.sync_copy(eidx_hbm.at[1, pl.ds(base, CHUNK)], rbufs[sl])
        pltpu.async_copy(nf_hbm.at[rbufs[sl]], jrows[sl], jsems[sl])
        pltpu.async_copy(nf_hbm.at[sbufs[sl]], irows[sl], isems[sl])

    def fin(i, sl):
        base = (w + NW * i) * CHUNK
        pltpu.make_async_copy(nf_hbm.at[pl.ds(0, CHUNK)], jrows[sl],
                              jsems[sl]).wait()
        pltpu.make_async_copy(nf_hbm.at[pl.ds(0, CHUNK)], irows[sl],
                              isems[sl]).wait()
        pltpu.sync_copy(jrows[sl], gj_hbm.at[pl.ds(base, CHUNK)])
        pltpu.sync_copy(irows[sl], gi_hbm.at[pl.ds(base, CHUNK)])

    load_idx(0, 0)
    load_idx(1, 1)

    def pair(i2, carry):
        a = 2 * i2
        fin(a, 0)
        load_idx(a + 2, 0)
        fin(a + 1, 1)

        @pl.when(a + 3 < BASE_H)
        def _():
            load_idx(a + 3, 1)

        return carry

    lax.fori_loop(0, (BASE_H - 1) // 2, pair, 0)
    fin(BASE_H - 1, 0)

    @pl.when(w < EXTRA_H)
    def _():
        load_idx(BASE_H, 0)
        fin(BASE_H, 0)


@functools.lru_cache(maxsize=None)
def _make_gather_k(half):
    return pl.kernel(
        functools.partial(_gather_body, half),
        mesh=_mesh(),
        out_type=(
            jax.ShapeDtypeStruct((EH, GW), jnp.float32),
            jax.ShapeDtypeStruct((EH, GW), jnp.float32),
        ),
        scratch_types=[
            pltpu.VMEM((CHUNK,), jnp.int32),
            pltpu.VMEM((CHUNK,), jnp.int32),
            pltpu.VMEM((CHUNK,), jnp.int32),
            pltpu.VMEM((CHUNK,), jnp.int32),
            pltpu.VMEM((CHUNK, GW), jnp.float32),
            pltpu.VMEM((CHUNK, GW), jnp.float32),
            pltpu.VMEM((CHUNK, GW), jnp.float32),
            pltpu.VMEM((CHUNK, GW), jnp.float32),
            pltpu.SemaphoreType.DMA,
            pltpu.SemaphoreType.DMA,
            pltpu.SemaphoreType.DMA,
            pltpu.SemaphoreType.DMA,
        ],
    )


TE = 640


def _edge_body(gj_ref, gi_ref, ea_ref, wcj_ref, wci_ref, we_ref,
               wce_ref, bcat_ref, wI_ref, bI_ref, wA_ref, bA_ref, wS_ref,
               bS_ref, mu_ref, p_ref):
    f32 = jnp.float32
    ef = jnp.dot(ea_ref[...], we_ref[...], preferred_element_type=f32)
    nfji = (jnp.dot(gj_ref[:, 0:F], wcj_ref[...], preferred_element_type=f32)
            + jnp.dot(gi_ref[:, 0:F], wci_ref[...], preferred_element_type=f32)
            + jnp.dot(ef, wce_ref[...], preferred_element_type=f32)
            + bcat_ref[...])
    vx = gi_ref[:, F:F + 1] - gj_ref[:, F:F + 1]
    vy = gi_ref[:, F + 1:F + 2] - gj_ref[:, F + 1:F + 2]
    vz = gi_ref[:, F + 2:F + 3] - gj_ref[:, F + 2:F + 3]
    lsq = vx * vx + vy * vy + vz * vz
    l = jnp.sqrt(lsq)
    inv = 1.0 / l
    nx = vx * inv
    ny = vy * inv
    nz = vz * inv
    d = jnp.exp(-l) - mu_ref[...]
    rbf = jnp.exp((-BETA) * d * d)
    phi = jnp.where(l < CUT, 0.5 * (jnp.cos((np.pi / CUT) * l) + 1.0), 0.0)
    q = rbf * phi
    f0I = (jnp.dot(q, wI_ref[...], preferred_element_type=f32) + bI_ref[...]) * phi
    f0A = (jnp.dot(q, wA_ref[...], preferred_element_type=f32) + bA_ref[...]) * phi
    f0S = (jnp.dot(q, wS_ref[...], preferred_element_type=f32) + bS_ref[...]) * phi
    gI = nfji * f0I
    gA = nfji * f0A
    gS = nfji * f0S
    t = 1.0 / 3.0
    m = [
        gI + gS * (nx * nx - t),
        gS * (nx * ny) - gA * nz,
        gS * (nx * nz) + gA * ny,
        gS * (nx * ny) + gA * nz,
        gI + gS * (ny * ny - t),
        gS * (ny * nz) - gA * nx,
        gS * (nx * nz) - gA * ny,
        gS * (ny * nz) + gA * nx,
        gI + gS * (nz * nz - t),
    ]
    for g in range(4):
        p_ref[g, :, 0:F] = m[2 * g]
        p_ref[g, :, F:PW] = m[2 * g + 1]
    p_ref[4, :, 0:F] = m[8]
    p_ref[4, :, F:PW] = jnp.zeros_like(m[8])


def _edge_stage(gj, gi, ea, wcj, wci, w_edge, wce, bcat, wI, bI, wA,
                bA, wS, bS, mu_row):
    n_blocks = EH // TE
    full = lambda shape: pl.BlockSpec(shape, lambda i: tuple(0 for _ in shape))
    return pl.pallas_call(
        _edge_body,
        grid=(n_blocks,),
        in_specs=[
            pl.BlockSpec((TE, GW), lambda i: (i, 0)),
            pl.BlockSpec((TE, GW), lambda i: (i, 0)),
            pl.BlockSpec((TE, EDGE_F), lambda i: (i, 0)),
            full((F, F)), full((F, F)), full((EDGE_F, F)), full((F, F)),
            full((1, F)), full((R, F)), full((1, F)), full((R, F)),
            full((1, F)), full((R, F)), full((1, F)), full((1, R)),
        ],
        out_specs=pl.BlockSpec((PG, TE, PW), lambda i: (0, i, 0)),
        out_shape=jax.ShapeDtypeStruct((PG, EH, PW), jnp.float32),
    )(gj, gi, ea, wcj, wci, w_edge, wce, bcat, wI, bI, wA, bA, wS, bS,
      mu_row)


def _scatter_body(half, eidx_hbm, p_hbm, out0_hbm, out1_hbm,
                  ib0, ib1, ub0, ub1, zbuf, acc, si0, si1, su0, su1):
    hbase = NCH_H * half
    c = lax.axis_index("c")
    s = lax.axis_index("s")
    w = c * NS + s
    ibufs, ubufs = (ib0, ib1), (ub0, ub1)
    isems, usems = (si0, si1), (su0, su1)
    zv = jnp.zeros((16,), jnp.float32)

    def zrow(r, carry):
        def zcol(j, carry2):
            zbuf[r, pl.ds(j * 16, 16)] = zv
            return carry2
        return lax.fori_loop(0, PW // 16, zcol, carry)

    lax.fori_loop(0, ZROWS, zrow, 0)

    for g in range(PG):
        plsc.subcore_barrier()
        for j in range(NPT // ZROWS):
            pltpu.sync_copy(zbuf, acc.at[pl.ds(s * NPT + j * ZROWS, ZROWS)])
        plsc.subcore_barrier()

        def start(i, sl):
            lbase = (w + NW * i) * CHUNK
            pltpu.async_copy(
                eidx_hbm.at[1, pl.ds(hbase * CHUNK + lbase, CHUNK)],
                ibufs[sl], isems[sl])
            pltpu.async_copy(p_hbm.at[g, pl.ds(lbase, CHUNK)], ubufs[sl],
                             usems[sl])

        def wait_scatter(sl):
            pltpu.make_async_copy(eidx_hbm.at[1, pl.ds(0, CHUNK)], ibufs[sl],
                                  isems[sl]).wait()
            pltpu.make_async_copy(p_hbm.at[g, pl.ds(0, CHUNK)], ubufs[sl],
                                  usems[sl]).wait()
            pltpu.sync_copy(ubufs[sl], acc.at[ibufs[sl]], add=True)

        start(0, 0)

        def pair(i2, carry):
            a = 2 * i2
            start(a + 1, 1)
            wait_scatter(0)
            start(a + 2, 0)
            wait_scatter(1)
            return carry

        lax.fori_loop(0, (BASE_H - 1) // 2, pair, 0)
        wait_scatter(0)

        @pl.when(w < EXTRA_H)
        def _():
            start(BASE_H, 1)
            wait_scatter(1)

        plsc.subcore_barrier()
        for j in range(NPT // ZROWS):
            rows = pl.ds(s * NPT + j * ZROWS, ZROWS)

            @pl.when(c == 0)
            def _():
                pltpu.sync_copy(acc.at[rows], out0_hbm.at[g, rows])

            @pl.when(c == 1)
            def _():
                pltpu.sync_copy(acc.at[rows], out1_hbm.at[g, rows])


@functools.lru_cache(maxsize=None)
def _make_scatter_k(half):
    return pl.kernel(
        functools.partial(_scatter_body, half),
        mesh=_mesh(),
        out_type=(
            jax.ShapeDtypeStruct((PG, NACC, PW), jnp.float32),
            jax.ShapeDtypeStruct((PG, NACC, PW), jnp.float32),
        ),
        scratch_types=[
            pltpu.VMEM((CHUNK,), jnp.int32),
            pltpu.VMEM((CHUNK,), jnp.int32),
            pltpu.VMEM((CHUNK, PW), jnp.float32),
            pltpu.VMEM((CHUNK, PW), jnp.float32),
            pltpu.VMEM((ZROWS, PW), jnp.float32),
            pltpu.VMEM_SHARED((NACC, PW), jnp.float32),
            pltpu.SemaphoreType.DMA,
            pltpu.SemaphoreType.DMA,
            pltpu.SemaphoreType.DMA,
            pltpu.SemaphoreType.DMA,
        ],
    )


TN = 400


def _tail_body(x0_ref, x1_ref, x2_ref, x3_ref, g_ref, b_ref, wm1_ref,
               bm1_ref, wm2_ref, bm2_ref, wIa_ref, wAa_ref, wSa_ref, o_ref):
    f32 = jnp.float32

    def comp(a, b):
        cc = 3 * a + b
        g, off = cc // 2, F * (cc % 2)
        return (x0_ref[g, :, off:off + F] + x1_ref[g, :, off:off + F]
                + x2_ref[g, :, off:off + F] + x3_ref[g, :, off:off + F])

    X00, X01, X02 = comp(0, 0), comp(0, 1), comp(0, 2)
    X10, X11, X12 = comp(1, 0), comp(1, 1), comp(1, 2)
    X20, X21, X22 = comp(2, 0), comp(2, 1), comp(2, 2)
    norm = (X00 * X00 + X01 * X01 + X02 * X02
            + X10 * X10 + X11 * X11 + X12 * X12
            + X20 * X20 + X21 * X21 + X22 * X22)
    mu = jnp.mean(norm, axis=-1, keepdims=True)
    dv = norm - mu
    var = jnp.mean(dv * dv, axis=-1, keepdims=True)
    h = dv * lax.rsqrt(var + 1e-5) * g_ref[...] + b_ref[...]
    h1 = jnp.dot(h, wm1_ref[...], preferred_element_type=f32) + bm1_ref[...]
    h1 = h1 * (1.0 / (1.0 + jnp.exp(-h1)))
    fs = jnp.dot(h1, wm2_ref[...], preferred_element_type=f32) + bm2_ref[...]
    fs = fs * (1.0 / (1.0 + jnp.exp(-fs)))
    fI = fs[:, 0 * F:1 * F]
    fA = fs[:, 1 * F:2 * F]
    fS = fs[:, 2 * F:3 * F]
    tr3 = (X00 + X11 + X22) * (1.0 / 3.0)
    wIa = wIa_ref[...]
    wAa = wAa_ref[...]
    wSa = wSa_ref[...]
    t1 = jnp.dot(tr3, wIa, preferred_element_type=f32)
    a01 = jnp.dot(0.5 * (X01 - X10), wAa, preferred_element_type=f32)
    a02 = jnp.dot(0.5 * (X02 - X20), wAa, preferred_element_type=f32)
    a12 = jnp.dot(0.5 * (X12 - X21), wAa, preferred_element_type=f32)
    s00 = jnp.dot(X00 - tr3, wSa, preferred_element_type=f32)
    s01 = jnp.dot(0.5 * (X01 + X10), wSa, preferred_element_type=f32)
    s02 = jnp.dot(0.5 * (X02 + X20), wSa, preferred_element_type=f32)
    s11 = jnp.dot(X11 - tr3, wSa, preferred_element_type=f32)
    s12 = jnp.dot(0.5 * (X12 + X21), wSa, preferred_element_type=f32)
    s22 = -(s00 + s11)
    o_ref[:, 0, :] = fI * t1 + fS * s00
    o_ref[:, 1, :] = fA * a01 + fS * s01
    o_ref[:, 2, :] = fA * a02 + fS * s02
    o_ref[:, 3, :] = -(fA * a01) + fS * s01
    o_ref[:, 4, :] = fI * t1 + fS * s11
    o_ref[:, 5, :] = fA * a12 + fS * s12
    o_ref[:, 6, :] = -(fA * a02) + fS * s02
    o_ref[:, 7, :] = -(fA * a12) + fS * s12
    o_ref[:, 8, :] = fI * t1 + fS * s22


def _tail(x0, x1, x2, x3, ln_g, ln_b, wm1, bm1, wm2p, bm2p, wIa, wAa, wSa):
    full = lambda shape: pl.BlockSpec(shape, lambda i: tuple(0 for _ in shape))
    return pl.pallas_call(
        _tail_body,
        grid=(N // TN,),
        in_specs=[
            pl.BlockSpec((PG, TN, PW), lambda i: (0, i, 0)),
            pl.BlockSpec((PG, TN, PW), lambda i: (0, i, 0)),
            pl.BlockSpec((PG, TN, PW), lambda i: (0, i, 0)),
            pl.BlockSpec((PG, TN, PW), lambda i: (0, i, 0)),
            full((1, F)), full((1, F)), full((F, F)), full((1, F)),
            full((F, 3 * F)), full((1, 3 * F)),
            full((F, F)), full((F, F)), full((F, F)),
        ],
        out_specs=pl.BlockSpec((TN, 9, F), lambda i: (i, 0, 0)),
        out_shape=jax.ShapeDtypeStruct((N, 9, F), jnp.float32),
    )(x0, x1, x2, x3, ln_g, ln_b, wm1, bm1, wm2p, bm2p, wIa, wAa, wSa)


def kernel(coordinates, node_attrs, edge_attrs, edge_index, num_nodes, W_node,
           W_edge, W_cat, b_cat, W_I, b_I, W_A, b_A, W_S, b_S, ln_g, ln_b,
           W_m1, b_m1, W_m2, b_m2, W_Ia, W_Aa, W_Sa):
    del num_nodes
    nf = _node_feats(node_attrs, W_node, coordinates)
    eidx = edge_index.astype(jnp.int32)

    wcj = W_cat[0:F]
    wci = W_cat[F:2 * F]
    wce = W_cat[2 * F:3 * F]
    mu_row = jnp.linspace(float(np.exp(-CUT)), 1.0, R,
                          dtype=jnp.float32).reshape(1, R)
    partials = []
    for half in (0, 1):
        gj, gi = _make_gather_k(half)(nf, eidx)
        ea_h = edge_attrs[half * EH:(half + 1) * EH]
        p = _edge_stage(gj, gi, ea_h, wcj, wci, W_edge, wce,
                        b_cat.reshape(1, F), W_I, b_I.reshape(1, F), W_A,
                        b_A.reshape(1, F), W_S, b_S.reshape(1, F), mu_row)
        partials.extend(_make_scatter_k(half)(eidx, p))
    x0, x1, x2, x3 = partials

    wm2p = W_m2.reshape(F, F, 3).transpose(0, 2, 1).reshape(F, 3 * F)
    bm2p = b_m2.reshape(F, 3).T.reshape(1, 3 * F)
    out = _tail(x0, x1, x2, x3, ln_g.reshape(1, F), ln_b.reshape(1, F), W_m1,
                b_m1.reshape(1, F), wm2p, bm2p, W_Ia, W_Aa, W_Sa)
    return out.transpose(0, 2, 1).reshape(N, F, 3, 3)
